# Initial kernel scaffold; baseline (speedup 1.0000x reference)
import jax
import jax.numpy as jnp
from jax.experimental import pallas as pl


def kernel(scores, linker_targets, candidate_lengths, cluster_ids):
    raise NotImplementedError("write your pallas kernel here")


# speedup vs baseline: 1.8291x; 1.8291x over previous
import jax
import jax.numpy as jnp
from jax import lax
from jax.experimental import pallas as pl
from jax.experimental.pallas import tpu as pltpu

_B, _M, _C = 2, 4096, 16
_W = _C + _M
_BR = 256
_BLOCKS_PER_BATCH = _M // _BR
_NBLK = _B * _BLOCKS_PER_BATCH


def _loss_kernel(scores_ref, cidpad_ref, cidrow_ref, len_ref, tgt_ref, out_ref):
    i = pl.program_id(0)

    s = scores_ref[...]
    col = lax.broadcasted_iota(jnp.int32, (_BR, _W), 1)
    lens = len_ref[0]
    is_coref = col >= _C
    valid = is_coref | (col < lens)

    rowmax = jnp.max(jnp.where(valid, s, -1e30), axis=1, keepdims=True)
    e = jnp.where(valid, jnp.exp(s - rowmax), 0.0)
    sum_all = jnp.sum(e, axis=1)

    cidp = cidpad_ref[0]
    cidr = cidrow_ref[0]
    row_in_batch = (i % _BLOCKS_PER_BATCH) * _BR + lax.broadcasted_iota(
        jnp.int32, (_BR, 1), 0)
    selfcol = row_in_batch + _C
    gold_coref = jnp.where(is_coref & (cidp == cidr) & (col != selfcol),
                           1.0, 0.0)

    tgt = tgt_ref[...].astype(jnp.float32)
    lin_w = tgt * (lax.broadcasted_iota(jnp.int32, (_BR, _C), 1)
                   < lens).astype(jnp.float32)
    e_lin = e[:, :_C]
    gsum_lin = jnp.sum(lin_w * e_lin, axis=1)
    cnt = jnp.sum(lin_w, axis=1) + jnp.sum(gold_coref, axis=1)

    self_w = jnp.where((cnt[:, None] == 0.0) & (col == selfcol), 1.0, 0.0)
    sum_gold = gsum_lin + jnp.sum((gold_coref + self_w) * e, axis=1)

    contrib = jnp.sum(jnp.log(sum_all) - jnp.log(sum_gold))

    @pl.when(i == 0)
    def _():
        out_ref[0, 0] = 0.0

    out_ref[0, 0] += contrib


@jax.jit
def kernel(scores, linker_targets, candidate_lengths, cluster_ids):
    s2 = scores.reshape(_B * _M, _W)
    tgt2 = linker_targets.reshape(_B * _M, _C)
    len3 = candidate_lengths.reshape(_NBLK, _BR, 1)
    cid3 = cluster_ids.reshape(_NBLK, _BR, 1)
    cidpad = jnp.concatenate(
        [jnp.full((_B, _C), -1, jnp.int32), cluster_ids],
        axis=1).reshape(_B, 1, _W)

    out = pl.pallas_call(
        _loss_kernel,
        grid=(_NBLK,),
        in_specs=[
            pl.BlockSpec((_BR, _W), lambda i: (i, 0)),
            pl.BlockSpec((1, 1, _W), lambda i: (i // _BLOCKS_PER_BATCH, 0, 0)),
            pl.BlockSpec((1, _BR, 1), lambda i: (i, 0, 0)),
            pl.BlockSpec((1, _BR, 1), lambda i: (i, 0, 0)),
            pl.BlockSpec((_BR, _C), lambda i: (i, 0)),
        ],
        out_specs=pl.BlockSpec(memory_space=pltpu.SMEM),
        out_shape=jax.ShapeDtypeStruct((1, 1), jnp.float32),
        compiler_params=pltpu.CompilerParams(
            dimension_semantics=("arbitrary",)),
    )(s2, cidpad, cid3, len3, tgt2)
    return out[0, 0]


# speedup vs baseline: 1.9478x; 1.0649x over previous
import jax
import jax.numpy as jnp
from jax import lax
from jax.experimental import pallas as pl
from jax.experimental.pallas import tpu as pltpu

_B, _M, _C = 2, 4096, 16
_W = _C + _M
_BR = 256
_BLOCKS_PER_BATCH = _M // _BR
_NBLK = _B * _BLOCKS_PER_BATCH


def _loss_kernel(scores_ref, cidpad_ref, cidrow_ref, len_ref, tgt_ref, out_ref):
    i = pl.program_id(0)

    s = scores_ref[...]
    rowmax = jnp.max(s, axis=1, keepdims=True)
    e = jnp.exp(s - rowmax)
    sum_full = jnp.sum(e, axis=1)

    cidp = cidpad_ref[0]
    cidr = cidrow_ref[0]
    d = cidp == cidr
    gsum_incl = jnp.sum(jnp.where(d, e, 0.0), axis=1)
    cnt_incl = jnp.sum(jnp.where(d, 1.0, 0.0), axis=1)

    lens = len_ref[0]
    e_lin = e[:, :_C]
    linmask = (lax.broadcasted_iota(jnp.int32, (_BR, _C), 1)
               < lens).astype(jnp.float32)
    lin_w = tgt_ref[...].astype(jnp.float32) * linmask
    sum_all = sum_full - jnp.sum((1.0 - linmask) * e_lin, axis=1)
    gsum_lin = jnp.sum(lin_w * e_lin, axis=1)
    cnt_lin = jnp.sum(lin_w, axis=1)

    start = pl.multiple_of((i % _BLOCKS_PER_BATCH) * _BR, 128)
    win = scores_ref[:, pl.ds(start, _BR + 128)]
    diagmask = (lax.broadcasted_iota(jnp.int32, (_BR, _BR + 128), 0) + _C
                == lax.broadcasted_iota(jnp.int32, (_BR, _BR + 128), 1))
    e_self = jnp.sum(jnp.where(diagmask, jnp.exp(win - rowmax), 0.0), axis=1)

    cnt = cnt_lin + cnt_incl - 1.0
    sum_gold = (gsum_lin + gsum_incl
                - jnp.where(cnt == 0.0, 0.0, 1.0) * e_self)
    contrib = jnp.sum(jnp.log(sum_all) - jnp.log(sum_gold))

    @pl.when(i == 0)
    def _():
        out_ref[0, 0] = 0.0

    out_ref[0, 0] += contrib


@jax.jit
def kernel(scores, linker_targets, candidate_lengths, cluster_ids):
    s2 = scores.reshape(_B * _M, _W)
    tgt2 = linker_targets.reshape(_B * _M, _C)
    len3 = candidate_lengths.reshape(_NBLK, _BR, 1)
    cid3 = cluster_ids.reshape(_NBLK, _BR, 1)
    cidpad = jnp.concatenate(
        [jnp.full((_B, _C), -1, jnp.int32), cluster_ids],
        axis=1).reshape(_B, 1, _W)

    out = pl.pallas_call(
        _loss_kernel,
        grid=(_NBLK,),
        in_specs=[
            pl.BlockSpec((_BR, _W), lambda i: (i, 0)),
            pl.BlockSpec((1, 1, _W), lambda i: (i // _BLOCKS_PER_BATCH, 0, 0)),
            pl.BlockSpec((1, _BR, 1), lambda i: (i, 0, 0)),
            pl.BlockSpec((1, _BR, 1), lambda i: (i, 0, 0)),
            pl.BlockSpec((_BR, _C), lambda i: (i, 0)),
        ],
        out_specs=pl.BlockSpec(memory_space=pltpu.SMEM),
        out_shape=jax.ShapeDtypeStruct((1, 1), jnp.float32),
        compiler_params=pltpu.CompilerParams(
            dimension_semantics=("arbitrary",)),
    )(s2, cidpad, cid3, len3, tgt2)
    return out[0, 0]
